# baseline (speedup 1.0000x reference)
import functools

import jax
import jax.numpy as jnp
from jax import lax
from jax.experimental import pallas as pl
from jax.experimental.pallas import tpu as pltpu
from jax.experimental.pallas import tpu_sc as plsc

_N = 10000
_E = 320000
_DIN = 128
_DOUT = 128
_DG = 64

_NS = 16
_NC = 2
_NW = _NS * _NC

_CH = 128
_CHS = 64
_EPT_A = 20480
_NCH_A = _EPT_A // _CH
_NCH_S = _EPT_A // _CHS
_EPAD_A = _NS * _EPT_A
_NACC = 10240

_RPT = 320
_NOUT_B = _NW * _RPT
_ICB = 8000
_WLCAP = 11264
_NEG = -1.0e30


def _bcast_last_(v):
    return lax.gather(
        v, jnp.full((16, 1), 15, jnp.int32),
        lax.GatherDimensionNumbers(offset_dims=(), collapsed_slice_dims=(0,),
                                   start_index_map=(0,)),
        (1,), mode=lax.GatherScatterMode.PROMISE_IN_BOUNDS)


def _leaky_(v):
    return jnp.where(v >= 0, v, 0.2 * v)


def _bn_(f, scale, offset):
    mean = jnp.mean(f, axis=1, keepdims=True)
    var = jnp.var(f, axis=1, keepdims=True) + 1e-09
    return (f - mean) * scale * lax.rsqrt(var) + offset


_BPRE = 2000


def _pre_body(x_ref, w0_ref, b0_ref, w1_ref, b1_ref, a1_ref, wpg_ref, wgm_ref,
              f0_ref, g0_ref, g1_ref, z_ref):
    xb = x_ref[...]
    f0 = jnp.maximum(jnp.dot(xb, w0_ref[...], preferred_element_type=jnp.float32)
                     + b0_ref[...], 0.0)
    f1 = jnp.maximum(jnp.dot(xb, w1_ref[...], preferred_element_type=jnp.float32)
                     + b1_ref[...], 0.0)
    an = _leaky_(jnp.dot(f1, a1_ref[...], preferred_element_type=jnp.float32))
    zj = jnp.dot(xb, wpg_ref[...], preferred_element_type=jnp.float32)
    p = jnp.dot(xb, wgm_ref[...], preferred_element_type=jnp.float32)
    f0_ref[...] = f0
    g0_ref[...] = f1
    g1_ref[...] = f1 * an
    z_ref[...] = jnp.concatenate(
        [zj, p, jnp.zeros((_BPRE, 128 - _DG - 1), jnp.float32)], axis=1)


def _tc_pre(x, w0, b0, w1, b1, a1, wpg, wgm):
    nblk = _N // _BPRE
    big = pl.BlockSpec((_BPRE, 128), lambda i: (i, 0))
    rep = lambda shp: pl.BlockSpec(shp, lambda i: (0, 0))
    return pl.pallas_call(
        _pre_body,
        grid=(nblk,),
        in_specs=[
            pl.BlockSpec((_BPRE, _DIN), lambda i: (i, 0)),
            rep((_DIN, _DOUT)), rep((1, _DOUT)),
            rep((_DIN, _DOUT)), rep((1, _DOUT)),
            rep((_DOUT, 1)), rep((_DIN, _DG)), rep((_DIN, 1)),
        ],
        out_specs=[big, big, big, big],
        out_shape=[jax.ShapeDtypeStruct((_N, 128), jnp.float32)] * 4,
    )(x, w0, b0, w1, b1, a1, wpg, wgm)


_MESH = plsc.VectorSubcoreMesh(core_axis_name="c", subcore_axis_name="s")


@functools.partial(
    pl.kernel,
    mesh=_MESH,
    compiler_params=pltpu.CompilerParams(needs_layout_passes=False),
    out_type=[jax.ShapeDtypeStruct((_N, 128), jnp.float32),
              jax.ShapeDtypeStruct((_N, 128), jnp.float32)],
    scratch_types=[
        pltpu.VMEM((8, _CHS), jnp.int32),
        pltpu.VMEM((8, _CHS), jnp.int32),
        pltpu.VMEM((_CHS, 128), jnp.float32),
        pltpu.VMEM((_CHS, 128), jnp.float32),
        pltpu.VMEM((_CHS, 128), jnp.float32),
        pltpu.VMEM((_CHS, 128), jnp.float32),
        pltpu.VMEM_SHARED((_NACC, 128), jnp.float32),
        pltpu.SemaphoreType.DMA,
        pltpu.SemaphoreType.DMA,
        pltpu.SemaphoreType.DMA,
        pltpu.SemaphoreType.DMA,
        pltpu.SemaphoreType.DMA,
        pltpu.SemaphoreType.DMA,
        pltpu.SemaphoreType.DMA,
        pltpu.SemaphoreType.DMA,
    ],
)
def _sc_sums(g0, g1, rowc, colc, zhbm, s1, s2, rows_v, cols_v, st0, st1, st2,
             st3, acc, gs0, gs1, gs2, gs3, ss0, ss1, ss2, ss3):
    c = lax.axis_index("c")
    s = lax.axis_index("s")

    if True:
        base = s * (_NACC // _NS)

        def _zacc(k, carry):
            pltpu.sync_copy(zhbm, acc.at[pl.ds(base + k * _CH, _CH)])
            return carry
        lax.fori_loop(0, _NACC // _NS // _CH, _zacc, 0)
        plsc.subcore_barrier()

        def _edge_loop(gsrc):
            sts = (st0, st1, st2, st3)
            gss = (gs0, gs1, gs2, gs3)
            sss = (ss0, ss1, ss2, ss3)

            def _blk(b, carry):
                pltpu.sync_copy(rowc.at[pl.ds(s * _NCH_S + b * 8, 8)], rows_v)
                pltpu.sync_copy(colc.at[pl.ds(s * _NCH_S + b * 8, 8)], cols_v)
                hg = [pltpu.async_copy(gsrc.at[cols_v.at[j]], sts[j], gss[j])
                      for j in range(4)]
                hs = [None] * 4
                for j in range(8):
                    p = j % 4
                    hg[p].wait()
                    hs[p] = pltpu.async_copy(sts[p], acc.at[rows_v.at[j]],
                                             sss[p], add=True)
                    if j + 4 < 8:
                        hs[p].wait()
                        hg[p] = pltpu.async_copy(gsrc.at[cols_v.at[j + 4]],
                                                 sts[p], gss[p])
                for p in range(4):
                    hs[p].wait()
                return carry
            lax.fori_loop(0, _NCH_S // 8, _blk, 0)

        @pl.when(c == 0)
        def _():
            _edge_loop(g0)

        @pl.when(c == 1)
        def _():
            _edge_loop(g1)

        plsc.subcore_barrier()

        ob = s * 624

        @pl.when(c == 0)
        def _():
            pltpu.sync_copy(acc.at[pl.ds(ob, 624)], s1.at[pl.ds(ob, 624)])

            @pl.when(s == 0)
            def _():
                pltpu.sync_copy(acc.at[pl.ds(9984, 16)], s1.at[pl.ds(9984, 16)])

        @pl.when(c == 1)
        def _():
            pltpu.sync_copy(acc.at[pl.ds(ob, 624)], s2.at[pl.ds(ob, 624)])

            @pl.when(s == 0)
            def _():
                pltpu.sync_copy(acc.at[pl.ds(9984, 16)], s2.at[pl.ds(9984, 16)])


_NCH_C = _EPAD_A // _CH // _NW


@functools.partial(
    pl.kernel,
    mesh=_MESH,
    compiler_params=pltpu.CompilerParams(needs_layout_passes=False),
    out_type=[jax.ShapeDtypeStruct((_N, 128), jnp.float32),
              jax.ShapeDtypeStruct((_N, 128), jnp.float32)],
    scratch_types=[
        pltpu.VMEM((8, _CH), jnp.int32),
        pltpu.VMEM((8, _CH), jnp.int32),
        pltpu.VMEM((8, _CH), jnp.float32),
        pltpu.VMEM((_CH, 128), jnp.float32),
        pltpu.VMEM((_CH, 128), jnp.float32),
        pltpu.VMEM_SHARED((_NACC, 128), jnp.float32),
        pltpu.SemaphoreType.DMA,
        pltpu.SemaphoreType.DMA,
        pltpu.SemaphoreType.DMA,
        pltpu.SemaphoreType.DMA,
    ],
)
def _sc_nmean(xt, rowc, colc, evc, zhbm, nm0, nm1, rows_v, cols_v, ev_v,
              st0, st1, acc, gs0, gs1, ss0, ss1):
    c = lax.axis_index("c")
    s = lax.axis_index("s")
    w = c * _NS + s
    lane = lax.iota(jnp.int32, 16)

    base = s * (_NACC // _NS)

    def _zacc(k, carry):
        pltpu.sync_copy(zhbm, acc.at[pl.ds(base + k * _CH, _CH)])
        return carry
    lax.fori_loop(0, _NACC // _NS // _CH, _zacc, 0)
    plsc.subcore_barrier()

    sts = (st0, st1)
    gss = (gs0, gs1)
    sss = (ss0, ss1)

    def _blk(b, carry):
        off = w * _NCH_C + b * 8
        pltpu.sync_copy(rowc.at[pl.ds(off, 8)], rows_v)
        pltpu.sync_copy(colc.at[pl.ds(off, 8)], cols_v)
        pltpu.sync_copy(evc.at[pl.ds(off, 8)], ev_v)
        hg = [pltpu.async_copy(xt.at[cols_v.at[j]], sts[j], gss[j])
              for j in range(2)]
        hs = [None, None]
        for j in range(8):
            p = j % 2
            hg[p].wait()
            jsp = jnp.full((16,), j, jnp.int32)
            stp = sts[p]

            def _scale(e, carry2, _jsp=jsp, _stp=stp):
                esp = jnp.full((16,), e, jnp.int32)
                evsp = plsc.load_gather(ev_v, [_jsp, esp])
                for k in range(8):
                    v = plsc.load_gather(_stp, [esp, lane + k * 16])
                    plsc.store_scatter(_stp, [esp, lane + k * 16], v * evsp)
                return carry2

            lax.fori_loop(0, _CH, _scale, 0)
            hs[p] = pltpu.async_copy(stp, acc.at[rows_v.at[j]], sss[p],
                                     add=True)
            if j + 2 < 8:
                hs[p].wait()
                hg[p] = pltpu.async_copy(xt.at[cols_v.at[j + 2]], sts[p],
                                         gss[p])
        hs[0].wait()
        hs[1].wait()
        return carry

    lax.fori_loop(0, _NCH_C // 8, _blk, 0)
    plsc.subcore_barrier()

    ob = s * 624

    @pl.when(c == 0)
    def _():
        pltpu.sync_copy(acc.at[pl.ds(ob, 624)], nm0.at[pl.ds(ob, 624)])

        @pl.when(s == 0)
        def _():
            pltpu.sync_copy(acc.at[pl.ds(9984, 16)], nm0.at[pl.ds(9984, 16)])

    @pl.when(c == 1)
    def _():
        pltpu.sync_copy(acc.at[pl.ds(ob, 624)], nm1.at[pl.ds(ob, 624)])

        @pl.when(s == 0)
        def _():
            pltpu.sync_copy(acc.at[pl.ds(9984, 16)], nm1.at[pl.ds(9984, 16)])


@functools.partial(
    pl.kernel,
    mesh=_MESH,
    compiler_params=pltpu.CompilerParams(needs_layout_passes=False),
    out_type=jax.ShapeDtypeStruct((_NOUT_B, 128), jnp.float32),
    scratch_types=[
        pltpu.VMEM((_ICB,), jnp.int32),
        pltpu.VMEM((_ICB,), jnp.int32),
        pltpu.VMEM((_ICB,), jnp.float32),
        pltpu.SemaphoreType.DMA,
        pltpu.VMEM((_WLCAP,), jnp.int32),
        pltpu.VMEM((_WLCAP,), jnp.int32),
        pltpu.VMEM((_WLCAP,), jnp.float32),
        pltpu.VMEM((_CH, 128), jnp.float32),
        pltpu.VMEM((_RPT, 128), jnp.float32),
        pltpu.SemaphoreType.DMA,
    ],
)
def _sc_maxscal(z, rowh, colh, evh, inith, out, rowb, colb, evb, semin, wlc,
                wlr, wle, stage, acc, sem):
    c = lax.axis_index("c")
    s = lax.axis_index("s")
    w = s * _NC + c
    lo = w * _RPT
    lane = lax.iota(jnp.int32, 16)

    pltpu.sync_copy(inith, acc)

    zi = jnp.zeros((16,), jnp.int32)

    def _zw(i, carry):
        wlc[pl.ds(i * 16, 16)] = zi
        return carry
    lax.fori_loop(0, _WLCAP // 16, _zw, 0)

    def _chunk(t, offc):
        base = t * _ICB
        h1 = pltpu.async_copy(rowh.at[pl.ds(base, _ICB)], rowb, semin)
        h2 = pltpu.async_copy(colh.at[pl.ds(base, _ICB)], colb, semin)
        h3 = pltpu.async_copy(evh.at[pl.ds(base, _ICB)], evb, semin)
        h1.wait()
        h2.wait()
        h3.wait()

        def _scan(j, off):
            rv = rowb[pl.ds(j * 16, 16)]
            cv = colb[pl.ds(j * 16, 16)]
            ev = evb[pl.ds(j * 16, 16)]
            m = (rv >= lo) & (rv < lo + _RPT)
            pref = jnp.cumsum(m.astype(jnp.int32))
            pos = off + pref - 1
            mm = m & (pos < _WLCAP)
            plsc.store_scatter(wlr, [pos], rv, mask=mm)
            plsc.store_scatter(wlc, [pos], cv, mask=mm)
            plsc.store_scatter(wle, [pos], ev, mask=mm)
            return off + _bcast_last_(pref)

        return lax.fori_loop(0, _ICB // 16, _scan, offc)

    offs = lax.fori_loop(0, _E // _ICB, _chunk, jnp.zeros((16,), jnp.int32))
    cnt = jnp.max(offs)

    def _sub(sc, carry):
        pltpu.async_copy(z.at[wlc.at[pl.ds(sc * _CH, _CH)]], stage, sem).wait()

        def _edge(e, carry2):
            evec = jnp.full((16,), e, jnp.int32)
            rsp = plsc.load_gather(wlr, [evec]) - lo
            jsp = evec - sc * _CH
            for k in range(4):
                zv = plsc.load_gather(stage, [jsp, lane + k * 16])
                av = plsc.load_gather(acc, [rsp, lane + k * 16])
                plsc.store_scatter(acc, [rsp, lane + k * 16],
                                   jnp.maximum(av, zv))
            evv = plsc.load_gather(wle, [evec])
            pv = plsc.load_gather(stage, [jsp, lane + 64])
            contrib = jnp.where(lane == 0, evv * pv,
                                jnp.where(lane == 1, 1.0, 0.0))
            a5 = plsc.load_gather(acc, [rsp, lane + 64])
            plsc.store_scatter(acc, [rsp, lane + 64], a5 + contrib)
            return carry2

        lax.fori_loop(sc * _CH, jnp.minimum(cnt, (sc + 1) * _CH), _edge, 0)
        return carry

    lax.fori_loop(0, (cnt + _CH - 1) // _CH, _sub, 0)

    pltpu.sync_copy(acc, out.at[pl.ds(lo, _RPT)])


def _post_body(x_ref, f0_ref, s1_ref, s2_ref, mxp_ref, nm0_ref, nm1_ref,
               a0_ref, wgx_ref, wgz_ref, wgm_ref, sc0_ref, of0_ref, sc1_ref,
               of1_ref, out_ref):
    xb = x_ref[...]
    f0 = f0_ref[...]
    att_self = _leaky_(jnp.dot(f0, a0_ref[...], preferred_element_type=jnp.float32))
    q = jnp.dot(xb, wgx_ref[...], preferred_element_type=jnp.float32)
    mxp = mxp_ref[...]
    mx = mxp[:, :_DG]
    deg = mxp[:, _DG + 1:_DG + 2]
    nmean = nm0_ref[...] + nm1_ref[...]
    t = jnp.dot(mx, wgz_ref[...], preferred_element_type=jnp.float32)
    sm = jnp.dot(nmean, wgm_ref[...], preferred_element_type=jnp.float32)
    gate = q + jnp.where(deg > 0, t, 0.0) + sm
    f1agg = gate * (att_self * s1_ref[...] + s2_ref[...])
    out_ref[...] = (_bn_(f0, sc0_ref[...], of0_ref[...]) +
                    _bn_(f1agg, sc1_ref[...], of1_ref[...]))


def _tc_post(x, f0, s1, s2, mxp, nm0, nm1, a0, wgx, wgz, wgm, sc0, of0,
             sc1, of1):
    nblk = _N // _BPRE
    big = pl.BlockSpec((_BPRE, 128), lambda i: (i, 0))
    rep = lambda shp: pl.BlockSpec(shp, lambda i: (0, 0))
    return pl.pallas_call(
        _post_body,
        grid=(nblk,),
        in_specs=[big, big, big, big, big, big, big,
                  rep((_DOUT, 1)), rep((_DIN, 1)), rep((_DG, 1)), rep((_DIN, 1)),
                  rep((1, _DOUT)), rep((1, _DOUT)), rep((1, _DOUT)), rep((1, _DOUT))],
        out_specs=big,
        out_shape=jax.ShapeDtypeStruct((_N, _DOUT), jnp.float32),
    )(x, f0, s1, s2, mxp, nm0, nm1, a0, wgx, wgz, wgm, sc0, of0, sc1, of1)


def kernel(x, edge_index, edge_vals, W0, b0, W1, b1, att0, offset0, scale0,
           offset1, scale1, weight_gate, weight_pool_gate):
    row = edge_index[0]
    col = edge_index[1]

    a0 = att0[0, :_DOUT].reshape(_DOUT, 1)
    a1 = att0[0, _DOUT:].reshape(_DOUT, 1)
    wgx = weight_gate[:_DIN].reshape(_DIN, 1)
    wgz = weight_gate[_DIN:_DIN + _DG].reshape(_DG, 1)
    wgm = weight_gate[_DIN + _DG:].reshape(_DIN, 1)

    f0, g0, g1, zt = _tc_pre(x, W0, b0.reshape(1, _DOUT), W1,
                             b1.reshape(1, _DOUT), a1, weight_pool_gate, wgm)

    npad = _EPAD_A - _E
    rowp = jnp.concatenate([row, jnp.full((npad,), _N, jnp.int32)])
    colp = jnp.concatenate([col, jnp.zeros((npad,), jnp.int32)])
    rowc = rowp.reshape(_EPAD_A // _CH, _CH)
    colc = colp.reshape(_EPAD_A // _CH, _CH)
    rowcs = rowp.reshape(_EPAD_A // _CHS, _CHS)
    colcs = colp.reshape(_EPAD_A // _CHS, _CHS)

    zhbm = jnp.zeros((_CH, 128), jnp.float32)
    inith = jnp.broadcast_to(
        jnp.concatenate([jnp.full((_DG,), _NEG, jnp.float32),
                         jnp.zeros((128 - _DG,), jnp.float32)])[None, :],
        (_RPT, 128))
    evp = jnp.concatenate([edge_vals, jnp.zeros((npad,), jnp.float32)])
    evc = evp.reshape(_EPAD_A // _CH, _CH)

    s1, s2 = _sc_sums(g0, g1, rowcs, colcs, zhbm)
    nm0, nm1 = _sc_nmean(x, rowc, colc, evc, zhbm)
    mxp = _sc_maxscal(zt, row, col, edge_vals, inith)

    return _tc_post(x, f0, s1, s2, mxp, nm0, nm1, a0, wgx, wgz, wgm,
                    scale0.reshape(1, _DOUT), offset0.reshape(1, _DOUT),
                    scale1.reshape(1, _DOUT), offset1.reshape(1, _DOUT))


# speedup vs baseline: 1.1041x; 1.1041x over previous
import functools

import jax
import jax.numpy as jnp
from jax import lax
from jax.experimental import pallas as pl
from jax.experimental.pallas import tpu as pltpu
from jax.experimental.pallas import tpu_sc as plsc

_N = 10000
_E = 320000
_DIN = 128
_DOUT = 128
_DG = 64

_NS = 16
_NC = 2
_NW = _NS * _NC

_CH = 128
_CHS = 64
_EPT_A = 20480
_NCH_A = _EPT_A // _CH
_NCH_S = _EPT_A // _CHS
_EPAD_A = _NS * _EPT_A
_NACC = 10240

_RPT = 320
_NOUT_B = _NW * _RPT
_ICB = 8000
_WLCAP = 11264
_NEG = -1.0e30


def _bcast_last_(v):
    return lax.gather(
        v, jnp.full((16, 1), 15, jnp.int32),
        lax.GatherDimensionNumbers(offset_dims=(), collapsed_slice_dims=(0,),
                                   start_index_map=(0,)),
        (1,), mode=lax.GatherScatterMode.PROMISE_IN_BOUNDS)


def _leaky_(v):
    return jnp.where(v >= 0, v, 0.2 * v)


def _bn_(f, scale, offset):
    mean = jnp.mean(f, axis=1, keepdims=True)
    var = jnp.var(f, axis=1, keepdims=True) + 1e-09
    return (f - mean) * scale * lax.rsqrt(var) + offset


_BPRE = 2000


def _pre_body(x_ref, w0_ref, b0_ref, w1_ref, b1_ref, a1_ref, wpg_ref, wgm_ref,
              f0_ref, g0_ref, g1_ref, z_ref):
    xb = x_ref[...]
    f0 = jnp.maximum(jnp.dot(xb, w0_ref[...], preferred_element_type=jnp.float32)
                     + b0_ref[...], 0.0)
    f1 = jnp.maximum(jnp.dot(xb, w1_ref[...], preferred_element_type=jnp.float32)
                     + b1_ref[...], 0.0)
    an = _leaky_(jnp.dot(f1, a1_ref[...], preferred_element_type=jnp.float32))
    zj = jnp.dot(xb, wpg_ref[...], preferred_element_type=jnp.float32)
    p = jnp.dot(xb, wgm_ref[...], preferred_element_type=jnp.float32)
    f0_ref[...] = f0
    g0_ref[...] = f1
    g1_ref[...] = f1 * an
    z_ref[...] = jnp.concatenate(
        [zj, p, jnp.zeros((_BPRE, 128 - _DG - 1), jnp.float32)], axis=1)


def _tc_pre(x, w0, b0, w1, b1, a1, wpg, wgm):
    nblk = _N // _BPRE
    big = pl.BlockSpec((_BPRE, 128), lambda i: (i, 0))
    rep = lambda shp: pl.BlockSpec(shp, lambda i: (0, 0))
    return pl.pallas_call(
        _pre_body,
        grid=(nblk,),
        in_specs=[
            pl.BlockSpec((_BPRE, _DIN), lambda i: (i, 0)),
            rep((_DIN, _DOUT)), rep((1, _DOUT)),
            rep((_DIN, _DOUT)), rep((1, _DOUT)),
            rep((_DOUT, 1)), rep((_DIN, _DG)), rep((_DIN, 1)),
        ],
        out_specs=[big, big, big, big],
        out_shape=[jax.ShapeDtypeStruct((_N, 128), jnp.float32)] * 4,
    )(x, w0, b0, w1, b1, a1, wpg, wgm)


_MESH = plsc.VectorSubcoreMesh(core_axis_name="c", subcore_axis_name="s")


@functools.partial(
    pl.kernel,
    mesh=_MESH,
    compiler_params=pltpu.CompilerParams(needs_layout_passes=False),
    out_type=[jax.ShapeDtypeStruct((_N, 128), jnp.float32),
              jax.ShapeDtypeStruct((_N, 128), jnp.float32)],
    scratch_types=[
        pltpu.VMEM((8, _CH), jnp.int32),
        pltpu.VMEM((8, _CH), jnp.int32),
        pltpu.VMEM((_CH, 128), jnp.float32),
        pltpu.VMEM((_CH, 128), jnp.float32),
        pltpu.VMEM_SHARED((_NACC, 128), jnp.float32),
        pltpu.SemaphoreType.DMA,
        pltpu.SemaphoreType.DMA,
        pltpu.SemaphoreType.DMA,
        pltpu.SemaphoreType.DMA,
    ],
)
def _sc_sums(g0, g1, rowc, colc, zhbm, s1, s2, rows_v, cols_v, st0, st1, acc,
             gs0, gs1, ss0, ss1):
    c = lax.axis_index("c")
    s = lax.axis_index("s")

    if True:
        base = s * (_NACC // _NS)

        def _zacc(k, carry):
            pltpu.sync_copy(zhbm, acc.at[pl.ds(base + k * _CH, _CH)])
            return carry
        lax.fori_loop(0, _NACC // _NS // _CH, _zacc, 0)
        plsc.subcore_barrier()

        def _edge_loop(gsrc):
            sts = (st0, st1)
            gss = (gs0, gs1)
            sss = (ss0, ss1)

            def _blk(b, carry):
                pltpu.sync_copy(rowc.at[pl.ds(s * _NCH_A + b * 8, 8)], rows_v)
                pltpu.sync_copy(colc.at[pl.ds(s * _NCH_A + b * 8, 8)], cols_v)
                hg = [pltpu.async_copy(gsrc.at[cols_v.at[j]], sts[j], gss[j])
                      for j in range(2)]
                hs = [None, None]
                for j in range(8):
                    p = j % 2
                    hg[p].wait()
                    hs[p] = pltpu.async_copy(sts[p], acc.at[rows_v.at[j]],
                                             sss[p], add=True)
                    if j + 2 < 8:
                        hs[p].wait()
                        hg[p] = pltpu.async_copy(gsrc.at[cols_v.at[j + 2]],
                                                 sts[p], gss[p])
                hs[0].wait()
                hs[1].wait()
                return carry
            lax.fori_loop(0, _NCH_A // 8, _blk, 0)

        @pl.when(c == 0)
        def _():
            _edge_loop(g0)

        @pl.when(c == 1)
        def _():
            _edge_loop(g1)

        plsc.subcore_barrier()

        ob = s * 624

        @pl.when(c == 0)
        def _():
            pltpu.sync_copy(acc.at[pl.ds(ob, 624)], s1.at[pl.ds(ob, 624)])

            @pl.when(s == 0)
            def _():
                pltpu.sync_copy(acc.at[pl.ds(9984, 16)], s1.at[pl.ds(9984, 16)])

        @pl.when(c == 1)
        def _():
            pltpu.sync_copy(acc.at[pl.ds(ob, 624)], s2.at[pl.ds(ob, 624)])

            @pl.when(s == 0)
            def _():
                pltpu.sync_copy(acc.at[pl.ds(9984, 16)], s2.at[pl.ds(9984, 16)])


_NCH_C = _EPAD_A // _CH // _NW


@functools.partial(
    pl.kernel,
    mesh=_MESH,
    compiler_params=pltpu.CompilerParams(needs_layout_passes=False),
    out_type=[jax.ShapeDtypeStruct((_N, 128), jnp.float32),
              jax.ShapeDtypeStruct((_N, 128), jnp.float32)],
    scratch_types=[
        pltpu.VMEM((8, _CH), jnp.int32),
        pltpu.VMEM((8, _CH), jnp.int32),
        pltpu.VMEM((8, _CH), jnp.float32),
        pltpu.VMEM((_CH, 128), jnp.float32),
        pltpu.VMEM((_CH, 128), jnp.float32),
        pltpu.VMEM_SHARED((_NACC, 128), jnp.float32),
        pltpu.SemaphoreType.DMA,
        pltpu.SemaphoreType.DMA,
        pltpu.SemaphoreType.DMA,
        pltpu.SemaphoreType.DMA,
    ],
)
def _sc_nmean(xt, rowc, colc, evc, zhbm, nm0, nm1, rows_v, cols_v, ev_v,
              st0, st1, acc, gs0, gs1, ss0, ss1):
    c = lax.axis_index("c")
    s = lax.axis_index("s")
    w = c * _NS + s
    lane = lax.iota(jnp.int32, 16)

    base = s * (_NACC // _NS)

    def _zacc(k, carry):
        pltpu.sync_copy(zhbm, acc.at[pl.ds(base + k * _CH, _CH)])
        return carry
    lax.fori_loop(0, _NACC // _NS // _CH, _zacc, 0)
    plsc.subcore_barrier()

    sts = (st0, st1)
    gss = (gs0, gs1)
    sss = (ss0, ss1)

    def _blk(b, carry):
        off = w * _NCH_C + b * 8
        pltpu.sync_copy(rowc.at[pl.ds(off, 8)], rows_v)
        pltpu.sync_copy(colc.at[pl.ds(off, 8)], cols_v)
        pltpu.sync_copy(evc.at[pl.ds(off, 8)], ev_v)
        hg = [pltpu.async_copy(xt.at[cols_v.at[j]], sts[j], gss[j])
              for j in range(2)]
        hs = [None, None]
        for j in range(8):
            p = j % 2
            hg[p].wait()
            jsp = jnp.full((16,), j, jnp.int32)
            stp = sts[p]

            def _scale(e, carry2, _jsp=jsp, _stp=stp):
                esp = jnp.full((16,), e, jnp.int32)
                evsp = plsc.load_gather(ev_v, [_jsp, esp])
                for k in range(8):
                    v = plsc.load_gather(_stp, [esp, lane + k * 16])
                    plsc.store_scatter(_stp, [esp, lane + k * 16], v * evsp)
                return carry2

            lax.fori_loop(0, _CH, _scale, 0)
            hs[p] = pltpu.async_copy(stp, acc.at[rows_v.at[j]], sss[p],
                                     add=True)
            if j + 2 < 8:
                hs[p].wait()
                hg[p] = pltpu.async_copy(xt.at[cols_v.at[j + 2]], sts[p],
                                         gss[p])
        hs[0].wait()
        hs[1].wait()
        return carry

    lax.fori_loop(0, _NCH_C // 8, _blk, 0)
    plsc.subcore_barrier()

    ob = s * 624

    @pl.when(c == 0)
    def _():
        pltpu.sync_copy(acc.at[pl.ds(ob, 624)], nm0.at[pl.ds(ob, 624)])

        @pl.when(s == 0)
        def _():
            pltpu.sync_copy(acc.at[pl.ds(9984, 16)], nm0.at[pl.ds(9984, 16)])

    @pl.when(c == 1)
    def _():
        pltpu.sync_copy(acc.at[pl.ds(ob, 624)], nm1.at[pl.ds(ob, 624)])

        @pl.when(s == 0)
        def _():
            pltpu.sync_copy(acc.at[pl.ds(9984, 16)], nm1.at[pl.ds(9984, 16)])


@functools.partial(
    pl.kernel,
    mesh=_MESH,
    compiler_params=pltpu.CompilerParams(needs_layout_passes=False),
    out_type=jax.ShapeDtypeStruct((_NOUT_B, 128), jnp.float32),
    scratch_types=[
        pltpu.VMEM((_ICB,), jnp.int32),
        pltpu.VMEM((_ICB,), jnp.int32),
        pltpu.VMEM((_ICB,), jnp.float32),
        pltpu.SemaphoreType.DMA,
        pltpu.VMEM((_WLCAP,), jnp.int32),
        pltpu.VMEM((_WLCAP,), jnp.int32),
        pltpu.VMEM((_WLCAP,), jnp.float32),
        pltpu.VMEM((_CH, 128), jnp.float32),
        pltpu.VMEM((_RPT, 128), jnp.float32),
        pltpu.SemaphoreType.DMA,
    ],
)
def _sc_maxscal(z, rowh, colh, evh, inith, out, rowb, colb, evb, semin, wlc,
                wlr, wle, stage, acc, sem):
    c = lax.axis_index("c")
    s = lax.axis_index("s")
    w = s * _NC + c
    lo = w * _RPT
    lane = lax.iota(jnp.int32, 16)

    pltpu.sync_copy(inith, acc)

    zi = jnp.zeros((16,), jnp.int32)

    def _zw(i, carry):
        wlc[pl.ds(i * 16, 16)] = zi
        return carry
    lax.fori_loop(0, _WLCAP // 16, _zw, 0)

    def _chunk(t, offc):
        base = t * _ICB
        h1 = pltpu.async_copy(rowh.at[pl.ds(base, _ICB)], rowb, semin)
        h2 = pltpu.async_copy(colh.at[pl.ds(base, _ICB)], colb, semin)
        h3 = pltpu.async_copy(evh.at[pl.ds(base, _ICB)], evb, semin)
        h1.wait()
        h2.wait()
        h3.wait()

        def _scan(j, off):
            rva = rowb[pl.ds(j * 32, 16)]
            rvb = rowb[pl.ds(j * 32 + 16, 16)]
            ma = (rva >= lo) & (rva < lo + _RPT)
            mb = (rvb >= lo) & (rvb < lo + _RPT)
            ca = jnp.cumsum(ma.astype(jnp.int32))
            cb = jnp.cumsum(mb.astype(jnp.int32))
            cva = colb[pl.ds(j * 32, 16)]
            cvb = colb[pl.ds(j * 32 + 16, 16)]
            eva = evb[pl.ds(j * 32, 16)]
            evb2 = evb[pl.ds(j * 32 + 16, 16)]
            offb = off + _bcast_last_(ca)
            pa = off + ca - 1
            pb = offb + cb - 1
            mma = ma & (pa < _WLCAP)
            mmb = mb & (pb < _WLCAP)
            plsc.store_scatter(wlr, [pa], rva, mask=mma)
            plsc.store_scatter(wlc, [pa], cva, mask=mma)
            plsc.store_scatter(wle, [pa], eva, mask=mma)
            plsc.store_scatter(wlr, [pb], rvb, mask=mmb)
            plsc.store_scatter(wlc, [pb], cvb, mask=mmb)
            plsc.store_scatter(wle, [pb], evb2, mask=mmb)
            return offb + _bcast_last_(cb)

        return lax.fori_loop(0, _ICB // 32, _scan, offc)

    offs = lax.fori_loop(0, _E // _ICB, _chunk, jnp.zeros((16,), jnp.int32))
    cnt = jnp.max(offs)

    def _sub(sc, carry):
        pltpu.async_copy(z.at[wlc.at[pl.ds(sc * _CH, _CH)]], stage, sem).wait()

        def _edge(e, carry2):
            evec = jnp.full((16,), e, jnp.int32)
            rsp = plsc.load_gather(wlr, [evec]) - lo
            jsp = evec - sc * _CH
            for k in range(4):
                zv = plsc.load_gather(stage, [jsp, lane + k * 16])
                av = plsc.load_gather(acc, [rsp, lane + k * 16])
                plsc.store_scatter(acc, [rsp, lane + k * 16],
                                   jnp.maximum(av, zv))
            evv = plsc.load_gather(wle, [evec])
            pv = plsc.load_gather(stage, [jsp, lane + 64])
            contrib = jnp.where(lane == 0, evv * pv,
                                jnp.where(lane == 1, 1.0, 0.0))
            a5 = plsc.load_gather(acc, [rsp, lane + 64])
            plsc.store_scatter(acc, [rsp, lane + 64], a5 + contrib)
            return carry2

        lax.fori_loop(sc * _CH, jnp.minimum(cnt, (sc + 1) * _CH), _edge, 0)
        return carry

    lax.fori_loop(0, (cnt + _CH - 1) // _CH, _sub, 0)

    pltpu.sync_copy(acc, out.at[pl.ds(lo, _RPT)])


def _post_body(x_ref, f0_ref, s1_ref, s2_ref, mxp_ref, nm0_ref, nm1_ref,
               a0_ref, wgx_ref, wgz_ref, wgm_ref, sc0_ref, of0_ref, sc1_ref,
               of1_ref, out_ref):
    xb = x_ref[...]
    f0 = f0_ref[...]
    att_self = _leaky_(jnp.dot(f0, a0_ref[...], preferred_element_type=jnp.float32))
    q = jnp.dot(xb, wgx_ref[...], preferred_element_type=jnp.float32)
    mxp = mxp_ref[...]
    mx = mxp[:, :_DG]
    deg = mxp[:, _DG + 1:_DG + 2]
    nmean = nm0_ref[...] + nm1_ref[...]
    t = jnp.dot(mx, wgz_ref[...], preferred_element_type=jnp.float32)
    sm = jnp.dot(nmean, wgm_ref[...], preferred_element_type=jnp.float32)
    gate = q + jnp.where(deg > 0, t, 0.0) + sm
    f1agg = gate * (att_self * s1_ref[...] + s2_ref[...])
    out_ref[...] = (_bn_(f0, sc0_ref[...], of0_ref[...]) +
                    _bn_(f1agg, sc1_ref[...], of1_ref[...]))


def _tc_post(x, f0, s1, s2, mxp, nm0, nm1, a0, wgx, wgz, wgm, sc0, of0,
             sc1, of1):
    nblk = _N // _BPRE
    big = pl.BlockSpec((_BPRE, 128), lambda i: (i, 0))
    rep = lambda shp: pl.BlockSpec(shp, lambda i: (0, 0))
    return pl.pallas_call(
        _post_body,
        grid=(nblk,),
        in_specs=[big, big, big, big, big, big, big,
                  rep((_DOUT, 1)), rep((_DIN, 1)), rep((_DG, 1)), rep((_DIN, 1)),
                  rep((1, _DOUT)), rep((1, _DOUT)), rep((1, _DOUT)), rep((1, _DOUT))],
        out_specs=big,
        out_shape=jax.ShapeDtypeStruct((_N, _DOUT), jnp.float32),
    )(x, f0, s1, s2, mxp, nm0, nm1, a0, wgx, wgz, wgm, sc0, of0, sc1, of1)


def kernel(x, edge_index, edge_vals, W0, b0, W1, b1, att0, offset0, scale0,
           offset1, scale1, weight_gate, weight_pool_gate):
    row = edge_index[0]
    col = edge_index[1]

    a0 = att0[0, :_DOUT].reshape(_DOUT, 1)
    a1 = att0[0, _DOUT:].reshape(_DOUT, 1)
    wgx = weight_gate[:_DIN].reshape(_DIN, 1)
    wgz = weight_gate[_DIN:_DIN + _DG].reshape(_DG, 1)
    wgm = weight_gate[_DIN + _DG:].reshape(_DIN, 1)

    f0, g0, g1, zt = _tc_pre(x, W0, b0.reshape(1, _DOUT), W1,
                             b1.reshape(1, _DOUT), a1, weight_pool_gate, wgm)

    npad = _EPAD_A - _E
    rowp = jnp.concatenate([row, jnp.full((npad,), _N, jnp.int32)])
    colp = jnp.concatenate([col, jnp.zeros((npad,), jnp.int32)])
    rowc = rowp.reshape(_EPAD_A // _CH, _CH)
    colc = colp.reshape(_EPAD_A // _CH, _CH)
    rowcs = rowp.reshape(_EPAD_A // _CHS, _CHS)
    colcs = colp.reshape(_EPAD_A // _CHS, _CHS)

    zhbm = jnp.zeros((_CH, 128), jnp.float32)
    inith = jnp.broadcast_to(
        jnp.concatenate([jnp.full((_DG,), _NEG, jnp.float32),
                         jnp.zeros((128 - _DG,), jnp.float32)])[None, :],
        (_RPT, 128))
    evp = jnp.concatenate([edge_vals, jnp.zeros((npad,), jnp.float32)])
    evc = evp.reshape(_EPAD_A // _CH, _CH)

    s1, s2 = _sc_sums(g0, g1, rowc, colc, zhbm)
    nm0, nm1 = _sc_nmean(x, rowc, colc, evc, zhbm)
    mxp = _sc_maxscal(zt, row, col, edge_vals, inith)

    return _tc_post(x, f0, s1, s2, mxp, nm0, nm1, a0, wgx, wgz, wgm,
                    scale0.reshape(1, _DOUT), offset0.reshape(1, _DOUT),
                    scale1.reshape(1, _DOUT), offset1.reshape(1, _DOUT))
